# baseline (speedup 1.0000x reference)
import functools

import jax
import jax.numpy as jnp
from jax import lax
from jax.experimental import pallas as pl
from jax.experimental.pallas import tpu as pltpu
from jax.experimental.pallas import tpu_sc as plsc

_NC = 2
_NS = 16
_L = 16
_NW = _NC * _NS

_CH = 8192
_G = 8192

_LN2 = 0.6931471805599453
_G1 = 0.43730217214327366
_G2 = -0.6729341930681538
_G3 = 0.31546760889308795
_G4 = -0.08001087690681215
_INV223 = float(2.0 ** -23)


def _log2_raw(v):
    bits = lax.bitcast_convert_type(v, jnp.int32)
    f = lax.bitcast_convert_type(
        jnp.bitwise_or(jnp.bitwise_and(bits, 0x007FFFFF), 0x3F800000),
        jnp.float32) - 1.0
    g = f * (_G1 + f * (_G2 + f * (_G3 + f * _G4)))
    return bits.astype(jnp.float32) * _INV223 + g


def _sc_partials(table, state_indices, choice_indices, weights, n_choices):
    n_obs = state_indices.shape[0]
    per_w = n_obs // _NW
    n_chunks = per_w // _CH
    n_half = n_chunks // 2

    mesh = plsc.VectorSubcoreMesh(
        core_axis_name="c", subcore_axis_name="s",
        num_cores=_NC, num_subcores=_NS)

    @functools.partial(
        pl.kernel,
        out_type=jax.ShapeDtypeStruct((_NW, 2, _L), jnp.float32),
        mesh=mesh,
        scratch_types=[
            pltpu.VMEM((per_w,), jnp.int32),
            pltpu.VMEM((per_w,), jnp.int32),
            pltpu.VMEM((per_w,), jnp.float32),
            pltpu.VMEM((_CH,), jnp.float32),
            pltpu.VMEM((_CH,), jnp.float32),
            pltpu.VMEM((2, _L), jnp.float32),
            pltpu.SemaphoreType.DMA,
            pltpu.SemaphoreType.DMA,
            pltpu.SemaphoreType.DMA,
        ],
    )
    def k(table_hbm, s_hbm, c_hbm, w_hbm, out_hbm, s_v, f_v, w_v, p0_v, p1_v,
          acc_v, sem_in, sem_a, sem_b):
        p_bufs = (p0_v, p1_v)
        wid = lax.axis_index("s") * _NC + lax.axis_index("c")
        base = wid * per_w

        loads = [
            pltpu.async_copy(s_hbm.at[pl.ds(base, per_w)], s_v, sem_in),
            pltpu.async_copy(c_hbm.at[pl.ds(base, per_w)], f_v, sem_in),
            pltpu.async_copy(w_hbm.at[pl.ds(base, per_w)], w_v, sem_in),
        ]
        for ld in loads:
            ld.wait()

        def flat_chunk(c):
            @plsc.parallel_loop(0, _CH, step=_L, unroll=8)
            def _(i):
                sl = pl.ds(pl.multiple_of(c * _CH + i, _L), _L)
                f_v[sl] = s_v[sl] * n_choices + f_v[sl]

        def gather_descs(c, buf, sem):
            return [
                pltpu.make_async_copy(
                    table_hbm.at[f_v.at[pl.ds(c * _CH + j * _G, _G)]],
                    p_bufs[buf].at[pl.ds(j * _G, _G)], sem)
                for j in range(_CH // _G)
            ]

        def fire(c, buf, sem):
            for d in gather_descs(c, buf, sem):
                d.start()

        def drain(c, buf, sem):
            for d in gather_descs(c, buf, sem):
                d.wait()

        def acc_chunk(c, buf, carry):
            @plsc.parallel_loop(0, _CH, step=4 * _L, carry=carry)
            def new_carry(i, cr):
                out = []
                for t in range(4):
                    a_ll, a_w = cr[2 * t], cr[2 * t + 1]
                    sl = pl.ds(pl.multiple_of(c * _CH + i + t * _L, _L), _L)
                    psl = pl.ds(pl.multiple_of(i + t * _L, _L), _L)
                    lw = w_v[sl]
                    out += [a_ll + lw * _log2_raw(p_bufs[buf][psl]), a_w + lw]
                return tuple(out)
            return new_carry

        flat_chunk(0)
        fire(0, 0, sem_a)
        flat_chunk(1)
        fire(1, 1, sem_b)

        def half_body(rr, carry):
            c0 = rr * 2
            @pl.when(rr < n_half - 1)
            def _():
                flat_chunk(c0 + 2)
            drain(c0, 0, sem_a)
            carry = acc_chunk(c0, 0, carry)

            @pl.when(rr < n_half - 1)
            def _():
                fire(c0 + 2, 0, sem_a)
                flat_chunk(c0 + 3)
            drain(c0 + 1, 1, sem_b)
            carry = acc_chunk(c0 + 1, 1, carry)

            @pl.when(rr < n_half - 1)
            def _():
                fire(c0 + 3, 1, sem_b)
            return carry

        z = jnp.zeros((_L,), jnp.float32)
        carry = lax.fori_loop(0, n_half, half_body, (z,) * 8)
        lane_raw = carry[0] + carry[2] + carry[4] + carry[6]
        lane_w = carry[1] + carry[3] + carry[5] + carry[7]
        acc_v[0, :] = lane_raw - 127.0 * lane_w
        acc_v[1, :] = lane_w
        pltpu.sync_copy(acc_v, out_hbm.at[wid])

    return k(table, state_indices, choice_indices, weights)


def kernel(choice_probs, state_indices, choice_indices, weights):
    n_choices = choice_probs.shape[1]
    table = choice_probs.reshape(-1)
    parts = _sc_partials(table, state_indices, choice_indices, weights,
                         n_choices)
    ll = _LN2 * jnp.sum(parts[:, 0, :])
    sw = jnp.sum(parts[:, 1, :])
    nll = -(ll / sw)
    return jnp.where(jnp.isfinite(nll), nll, jnp.asarray(1e10, nll.dtype))


# speedup vs baseline: 1.0234x; 1.0234x over previous
import functools

import jax
import jax.numpy as jnp
from jax import lax
from jax.experimental import pallas as pl
from jax.experimental.pallas import tpu as pltpu
from jax.experimental.pallas import tpu_sc as plsc

_NC = 2
_NS = 16
_L = 16
_NW = _NC * _NS

_CH = 2048
_G = 2048

_LN2 = 0.6931471805599453
_G1 = 0.43730217214327366
_G2 = -0.6729341930681538
_G3 = 0.31546760889308795
_G4 = -0.08001087690681215
_INV223 = float(2.0 ** -23)


def _log2_raw(v):
    bits = lax.bitcast_convert_type(v, jnp.int32)
    f = lax.bitcast_convert_type(
        jnp.bitwise_or(jnp.bitwise_and(bits, 0x007FFFFF), 0x3F800000),
        jnp.float32) - 1.0
    g = f * (_G1 + f * (_G2 + f * (_G3 + f * _G4)))
    return bits.astype(jnp.float32) * _INV223 + g


def _sc_partials(table, state_indices, choice_indices, weights, n_choices):
    n_obs = state_indices.shape[0]
    per_w = n_obs // _NW
    n_chunks = per_w // _CH
    n_half = n_chunks // 2

    mesh = plsc.VectorSubcoreMesh(
        core_axis_name="c", subcore_axis_name="s",
        num_cores=_NC, num_subcores=_NS)

    @functools.partial(
        pl.kernel,
        out_type=jax.ShapeDtypeStruct((_NW, 2, _L), jnp.float32),
        mesh=mesh,
        scratch_types=[
            pltpu.VMEM((per_w,), jnp.int32),
            pltpu.VMEM((per_w,), jnp.int32),
            pltpu.VMEM((per_w,), jnp.float32),
            pltpu.VMEM((_CH,), jnp.float32),
            pltpu.VMEM((_CH,), jnp.float32),
            pltpu.VMEM((2, _L), jnp.float32),
            pltpu.SemaphoreType.DMA,
            pltpu.SemaphoreType.DMA,
            pltpu.SemaphoreType.DMA,
        ],
    )
    def k(table_hbm, s_hbm, c_hbm, w_hbm, out_hbm, s_v, f_v, w_v, p0_v, p1_v,
          acc_v, sem_in, sem_a, sem_b):
        p_bufs = (p0_v, p1_v)
        wid = lax.axis_index("s") * _NC + lax.axis_index("c")
        base = wid * per_w

        loads = [
            pltpu.async_copy(s_hbm.at[pl.ds(base, per_w)], s_v, sem_in),
            pltpu.async_copy(c_hbm.at[pl.ds(base, per_w)], f_v, sem_in),
            pltpu.async_copy(w_hbm.at[pl.ds(base, per_w)], w_v, sem_in),
        ]
        for ld in loads:
            ld.wait()

        def flat_chunk(c):
            @plsc.parallel_loop(0, _CH, step=_L, unroll=8)
            def _(i):
                sl = pl.ds(pl.multiple_of(c * _CH + i, _L), _L)
                f_v[sl] = s_v[sl] * n_choices + f_v[sl]

        def gather_descs(c, buf, sem):
            return [
                pltpu.make_async_copy(
                    table_hbm.at[f_v.at[pl.ds(c * _CH + j * _G, _G)]],
                    p_bufs[buf].at[pl.ds(j * _G, _G)], sem)
                for j in range(_CH // _G)
            ]

        def fire(c, buf, sem):
            for d in gather_descs(c, buf, sem):
                d.start()

        def drain(c, buf, sem):
            for d in gather_descs(c, buf, sem):
                d.wait()

        def acc_chunk(c, buf, carry):
            @plsc.parallel_loop(0, _CH, step=4 * _L, carry=carry)
            def new_carry(i, cr):
                out = []
                for t in range(4):
                    a_ll, a_w = cr[2 * t], cr[2 * t + 1]
                    sl = pl.ds(pl.multiple_of(c * _CH + i + t * _L, _L), _L)
                    psl = pl.ds(pl.multiple_of(i + t * _L, _L), _L)
                    lw = w_v[sl]
                    out += [a_ll + lw * _log2_raw(p_bufs[buf][psl]), a_w + lw]
                return tuple(out)
            return new_carry

        flat_chunk(0)
        fire(0, 0, sem_a)
        flat_chunk(1)
        fire(1, 1, sem_b)

        def half_body(rr, carry):
            c0 = rr * 2
            @pl.when(rr < n_half - 1)
            def _():
                flat_chunk(c0 + 2)
            drain(c0, 0, sem_a)
            carry = acc_chunk(c0, 0, carry)

            @pl.when(rr < n_half - 1)
            def _():
                fire(c0 + 2, 0, sem_a)
                flat_chunk(c0 + 3)
            drain(c0 + 1, 1, sem_b)
            carry = acc_chunk(c0 + 1, 1, carry)

            @pl.when(rr < n_half - 1)
            def _():
                fire(c0 + 3, 1, sem_b)
            return carry

        z = jnp.zeros((_L,), jnp.float32)
        carry = lax.fori_loop(0, n_half, half_body, (z,) * 8)
        lane_raw = carry[0] + carry[2] + carry[4] + carry[6]
        lane_w = carry[1] + carry[3] + carry[5] + carry[7]
        acc_v[0, :] = lane_raw - 127.0 * lane_w
        acc_v[1, :] = lane_w
        pltpu.sync_copy(acc_v, out_hbm.at[wid])

    return k(table, state_indices, choice_indices, weights)


def kernel(choice_probs, state_indices, choice_indices, weights):
    n_choices = choice_probs.shape[1]
    table = choice_probs.reshape(-1)
    parts = _sc_partials(table, state_indices, choice_indices, weights,
                         n_choices)
    ll = _LN2 * jnp.sum(parts[:, 0, :])
    sw = jnp.sum(parts[:, 1, :])
    nll = -(ll / sw)
    return jnp.where(jnp.isfinite(nll), nll, jnp.asarray(1e10, nll.dtype))


# speedup vs baseline: 1.0306x; 1.0070x over previous
import functools

import jax
import jax.numpy as jnp
from jax import lax
from jax.experimental import pallas as pl
from jax.experimental.pallas import tpu as pltpu
from jax.experimental.pallas import tpu_sc as plsc

_NC = 2
_NS = 16
_L = 16
_NW = _NC * _NS

_CH = 4096
_G = 4096

_LN2 = 0.6931471805599453
_G1 = 0.417299452916571
_G2 = -0.5729205969346398
_G3 = 0.15544585507946448
_INV223 = float(2.0 ** -23)


def _log2_raw(v):
    bits = lax.bitcast_convert_type(v, jnp.int32)
    f = lax.bitcast_convert_type(
        jnp.bitwise_or(jnp.bitwise_and(bits, 0x007FFFFF), 0x3F800000),
        jnp.float32) - 1.0
    g = f * (_G1 + f * (_G2 + f * _G3))
    return bits.astype(jnp.float32) * _INV223 + g


def _sc_partials(table, state_indices, choice_indices, weights, n_choices):
    n_obs = state_indices.shape[0]
    per_w = n_obs // _NW
    n_chunks = per_w // _CH
    n_half = n_chunks // 2

    mesh = plsc.VectorSubcoreMesh(
        core_axis_name="c", subcore_axis_name="s",
        num_cores=_NC, num_subcores=_NS)

    @functools.partial(
        pl.kernel,
        out_type=jax.ShapeDtypeStruct((_NW, 2, _L), jnp.float32),
        mesh=mesh,
        scratch_types=[
            pltpu.VMEM((per_w,), jnp.int32),
            pltpu.VMEM((per_w,), jnp.int32),
            pltpu.VMEM((per_w,), jnp.float32),
            pltpu.VMEM((_CH,), jnp.float32),
            pltpu.VMEM((_CH,), jnp.float32),
            pltpu.VMEM((2, _L), jnp.float32),
            pltpu.SemaphoreType.DMA,
            pltpu.SemaphoreType.DMA,
            pltpu.SemaphoreType.DMA,
        ],
    )
    def k(table_hbm, s_hbm, c_hbm, w_hbm, out_hbm, s_v, f_v, w_v, p0_v, p1_v,
          acc_v, sem_in, sem_a, sem_b):
        p_bufs = (p0_v, p1_v)
        wid = lax.axis_index("s") * _NC + lax.axis_index("c")
        base = wid * per_w

        loads = [
            pltpu.async_copy(s_hbm.at[pl.ds(base, per_w)], s_v, sem_in),
            pltpu.async_copy(c_hbm.at[pl.ds(base, per_w)], f_v, sem_in),
            pltpu.async_copy(w_hbm.at[pl.ds(base, per_w)], w_v, sem_in),
        ]
        for ld in loads:
            ld.wait()

        def flat_chunk(c):
            @plsc.parallel_loop(0, _CH, step=_L, unroll=8)
            def _(i):
                sl = pl.ds(pl.multiple_of(c * _CH + i, _L), _L)
                f_v[sl] = s_v[sl] * n_choices + f_v[sl]

        def gather_descs(c, buf, sem):
            return [
                pltpu.make_async_copy(
                    table_hbm.at[f_v.at[pl.ds(c * _CH + j * _G, _G)]],
                    p_bufs[buf].at[pl.ds(j * _G, _G)], sem)
                for j in range(_CH // _G)
            ]

        def fire(c, buf, sem):
            for d in gather_descs(c, buf, sem):
                d.start()

        def drain(c, buf, sem):
            for d in gather_descs(c, buf, sem):
                d.wait()

        def acc_chunk(c, buf, carry):
            @plsc.parallel_loop(0, _CH, step=4 * _L, carry=carry)
            def new_carry(i, cr):
                out = []
                for t in range(4):
                    a_ll, a_w = cr[2 * t], cr[2 * t + 1]
                    sl = pl.ds(pl.multiple_of(c * _CH + i + t * _L, _L), _L)
                    psl = pl.ds(pl.multiple_of(i + t * _L, _L), _L)
                    lw = w_v[sl]
                    out += [a_ll + lw * _log2_raw(p_bufs[buf][psl]), a_w + lw]
                return tuple(out)
            return new_carry

        flat_chunk(0)
        fire(0, 0, sem_a)
        flat_chunk(1)
        fire(1, 1, sem_b)

        def half_body(rr, carry):
            c0 = rr * 2
            @pl.when(rr < n_half - 1)
            def _():
                flat_chunk(c0 + 2)
            drain(c0, 0, sem_a)
            carry = acc_chunk(c0, 0, carry)

            @pl.when(rr < n_half - 1)
            def _():
                fire(c0 + 2, 0, sem_a)
                flat_chunk(c0 + 3)
            drain(c0 + 1, 1, sem_b)
            carry = acc_chunk(c0 + 1, 1, carry)

            @pl.when(rr < n_half - 1)
            def _():
                fire(c0 + 3, 1, sem_b)
            return carry

        z = jnp.zeros((_L,), jnp.float32)
        carry = lax.fori_loop(0, n_half, half_body, (z,) * 8)
        lane_raw = carry[0] + carry[2] + carry[4] + carry[6]
        lane_w = carry[1] + carry[3] + carry[5] + carry[7]
        acc_v[0, :] = lane_raw - 127.0 * lane_w
        acc_v[1, :] = lane_w
        pltpu.sync_copy(acc_v, out_hbm.at[wid])

    return k(table, state_indices, choice_indices, weights)


def kernel(choice_probs, state_indices, choice_indices, weights):
    n_choices = choice_probs.shape[1]
    table = choice_probs.reshape(-1)
    parts = _sc_partials(table, state_indices, choice_indices, weights,
                         n_choices)
    ll = _LN2 * jnp.sum(parts[:, 0, :])
    sw = jnp.sum(parts[:, 1, :])
    nll = -(ll / sw)
    return jnp.where(jnp.isfinite(nll), nll, jnp.asarray(1e10, nll.dtype))


# speedup vs baseline: 1.0381x; 1.0073x over previous
import functools

import jax
import jax.numpy as jnp
from jax import lax
from jax.experimental import pallas as pl
from jax.experimental.pallas import tpu as pltpu
from jax.experimental.pallas import tpu_sc as plsc

_NC = 2
_NS = 16
_L = 16
_NW = _NC * _NS

_CH = 4096
_G = 4096

_LN2 = 0.6931471805599453
_G1 = 0.44159207720654886
_G2 = -0.7072534335743554
_G3 = 0.41156148231045137
_G4 = -0.18983244652665532
_G5 = 0.043928627847937184
_INV223 = float(2.0 ** -23)
_TBITS = 10
_TSIZE = 1 << _TBITS


def _g_of(f):
    return f * (_G1 + f * (_G2 + f * (_G3 + f * (_G4 + f * _G5))))


def _log2_raw(v, tab_ref):
    bits = lax.bitcast_convert_type(v, jnp.int32)
    idx = jnp.bitwise_and(lax.shift_right_logical(bits, 23 - _TBITS),
                          _TSIZE - 1)
    g = plsc.load_gather(tab_ref, [idx])
    return bits.astype(jnp.float32) * _INV223 + g


def _sc_partials(table, state_indices, choice_indices, weights, n_choices):
    n_obs = state_indices.shape[0]
    per_w = n_obs // _NW
    n_chunks = per_w // _CH
    n_half = n_chunks // 2

    mesh = plsc.VectorSubcoreMesh(
        core_axis_name="c", subcore_axis_name="s",
        num_cores=_NC, num_subcores=_NS)

    @functools.partial(
        pl.kernel,
        out_type=jax.ShapeDtypeStruct((_NW, 2, _L), jnp.float32),
        mesh=mesh,
        compiler_params=pltpu.CompilerParams(needs_layout_passes=False),
        scratch_types=[
            pltpu.VMEM((per_w,), jnp.int32),
            pltpu.VMEM((per_w,), jnp.int32),
            pltpu.VMEM((per_w,), jnp.float32),
            pltpu.VMEM((_CH,), jnp.float32),
            pltpu.VMEM((_CH,), jnp.float32),
            pltpu.VMEM((2, _L), jnp.float32),
            pltpu.VMEM((_TSIZE,), jnp.float32),
            pltpu.SemaphoreType.DMA,
            pltpu.SemaphoreType.DMA,
            pltpu.SemaphoreType.DMA,
        ],
    )
    def k(table_hbm, s_hbm, c_hbm, w_hbm, out_hbm, s_v, f_v, w_v, p0_v, p1_v,
          acc_v, tab_v, sem_in, sem_a, sem_b):
        p_bufs = (p0_v, p1_v)
        wid = lax.axis_index("s") * _NC + lax.axis_index("c")
        base = wid * per_w

        loads = [
            pltpu.async_copy(s_hbm.at[pl.ds(base, per_w)], s_v, sem_in),
            pltpu.async_copy(c_hbm.at[pl.ds(base, per_w)], f_v, sem_in),
            pltpu.async_copy(w_hbm.at[pl.ds(base, per_w)], w_v, sem_in),
        ]

        lane = lax.iota(jnp.int32, _L)

        @plsc.parallel_loop(0, _TSIZE, step=_L)
        def _(j):
            fmid = ((lane + j).astype(jnp.float32) + 0.5) * (1.0 / _TSIZE)
            tab_v[pl.ds(pl.multiple_of(j, _L), _L)] = _g_of(fmid)

        for ld in loads:
            ld.wait()

        def flat_chunk(c):
            @plsc.parallel_loop(0, _CH, step=_L, unroll=8)
            def _(i):
                sl = pl.ds(pl.multiple_of(c * _CH + i, _L), _L)
                f_v[sl] = s_v[sl] * n_choices + f_v[sl]

        def gather_descs(c, buf, sem):
            return [
                pltpu.make_async_copy(
                    table_hbm.at[f_v.at[pl.ds(c * _CH + j * _G, _G)]],
                    p_bufs[buf].at[pl.ds(j * _G, _G)], sem)
                for j in range(_CH // _G)
            ]

        def fire(c, buf, sem):
            for d in gather_descs(c, buf, sem):
                d.start()

        def drain(c, buf, sem):
            for d in gather_descs(c, buf, sem):
                d.wait()

        def acc_chunk(c, buf, carry):
            @plsc.parallel_loop(0, _CH, step=4 * _L, carry=carry)
            def new_carry(i, cr):
                out = []
                for t in range(4):
                    a_ll, a_w = cr[2 * t], cr[2 * t + 1]
                    sl = pl.ds(pl.multiple_of(c * _CH + i + t * _L, _L), _L)
                    psl = pl.ds(pl.multiple_of(i + t * _L, _L), _L)
                    lw = w_v[sl]
                    out += [a_ll + lw * _log2_raw(p_bufs[buf][psl], tab_v),
                            a_w + lw]
                return tuple(out)
            return new_carry

        flat_chunk(0)
        fire(0, 0, sem_a)
        flat_chunk(1)
        fire(1, 1, sem_b)

        def half_body(rr, carry):
            c0 = rr * 2
            @pl.when(rr < n_half - 1)
            def _():
                flat_chunk(c0 + 2)
            drain(c0, 0, sem_a)
            carry = acc_chunk(c0, 0, carry)

            @pl.when(rr < n_half - 1)
            def _():
                fire(c0 + 2, 0, sem_a)
                flat_chunk(c0 + 3)
            drain(c0 + 1, 1, sem_b)
            carry = acc_chunk(c0 + 1, 1, carry)

            @pl.when(rr < n_half - 1)
            def _():
                fire(c0 + 3, 1, sem_b)
            return carry

        z = jnp.zeros((_L,), jnp.float32)
        carry = lax.fori_loop(0, n_half, half_body, (z,) * 8)
        lane_raw = carry[0] + carry[2] + carry[4] + carry[6]
        lane_w = carry[1] + carry[3] + carry[5] + carry[7]
        acc_v[0, :] = lane_raw - 127.0 * lane_w
        acc_v[1, :] = lane_w
        pltpu.sync_copy(acc_v, out_hbm.at[wid])

    return k(table, state_indices, choice_indices, weights)


def kernel(choice_probs, state_indices, choice_indices, weights):
    n_choices = choice_probs.shape[1]
    table = choice_probs.reshape(-1)
    parts = _sc_partials(table, state_indices, choice_indices, weights,
                         n_choices)
    ll = _LN2 * jnp.sum(parts[:, 0, :])
    sw = jnp.sum(parts[:, 1, :])
    nll = -(ll / sw)
    return jnp.where(jnp.isfinite(nll), nll, jnp.asarray(1e10, nll.dtype))
